# baseline (speedup 1.0000x reference)
import functools

import jax
import jax.numpy as jnp
from jax import lax
from jax.experimental import pallas as pl
from jax.experimental.pallas import tpu as pltpu
from jax.experimental.pallas import tpu_sc as plsc

B = 16384
D = 64
V = 1000000

_NC = 2
_NS = 16
_NW = _NC * _NS
_BPW = B // _NW
_NGRP = _BPW // 16

_NFULL = V // 128
_TAIL = V - _NFULL * 128
_BLKS_PER_W = _NFULL // _NW + 1


def _sc_transpose_body(wt_hbm, out_hbm, blk_in, blk_out):
    wid = lax.axis_index("s") * _NC + lax.axis_index("c")
    lane = lax.iota(jnp.int32, 16)

    def block(i, carry):
        k = i * _NW + wid

        @pl.when(k < _NFULL)
        def _():
            off = pl.multiple_of(k * 128, 128)
            pltpu.sync_copy(wt_hbm.at[:, pl.ds(off, 128)], blk_in)
            for c in range(D):
                for l0 in range(0, 128, 16):
                    vals = blk_in[c, pl.ds(l0, 16)]
                    plsc.store_scatter(
                        blk_out, [l0 + lane, jnp.full((16,), c, jnp.int32)],
                        vals)
            pltpu.sync_copy(blk_out, out_hbm.at[pl.ds(off, 128)])

        return carry

    lax.fori_loop(0, _BLKS_PER_W, block, 0)


@functools.lru_cache(maxsize=1)
def _sc_transpose():
    return pl.kernel(
        _sc_transpose_body,
        out_type=jax.ShapeDtypeStruct((_NFULL * 128, D), jnp.float32),
        mesh=plsc.VectorSubcoreMesh(core_axis_name="c", subcore_axis_name="s"),
        compiler_params=pltpu.CompilerParams(needs_layout_passes=False),
        scratch_types=[
            pltpu.VMEM((D, 128), jnp.float32),
            pltpu.VMEM((128, D), jnp.float32),
        ],
    )


def _sc_rowdma_body(idx_hbm, tab_hbm, out_hbm, idx_v, rows_v, sem):
    wid = lax.axis_index("s") * _NC + lax.axis_index("c")
    base = wid * _BPW
    pltpu.sync_copy(idx_hbm.at[pl.ds(base, _BPW)], idx_v)

    def group(g, carry):
        chunk = idx_v[pl.ds(g * 16, 16)]
        for j in range(16):
            s = chunk[j]
            pltpu.async_copy(tab_hbm.at[pl.ds(s, 1)],
                             rows_v.at[pl.ds(g * 16 + j, 1)], sem)
        return carry

    lax.fori_loop(0, _NGRP, group, 0)
    pltpu.make_async_copy(tab_hbm.at[pl.ds(0, _BPW)], rows_v, sem).wait()
    pltpu.sync_copy(rows_v, out_hbm.at[pl.ds(base, _BPW)])


def _sc_rowdma_tail_body(idx_hbm, tab_hbm, tail_hbm, out_hbm,
                         idx_v, rows_v, sem):
    wid = lax.axis_index("s") * _NC + lax.axis_index("c")
    base = wid * _BPW
    pltpu.sync_copy(idx_hbm.at[pl.ds(base, _BPW)], idx_v)

    def group(g, carry):
        chunk = idx_v[pl.ds(g * 16, 16)]
        for j in range(16):
            s = chunk[j]
            r = g * 16 + j

            @pl.when(s < _NFULL * 128)
            def _():
                pltpu.async_copy(tab_hbm.at[pl.ds(s, 1)],
                                 rows_v.at[pl.ds(r, 1)], sem)

            @pl.when(s >= _NFULL * 128)
            def _():
                pltpu.async_copy(tail_hbm.at[pl.ds(s - _NFULL * 128, 1)],
                                 rows_v.at[pl.ds(r, 1)], sem)
        return carry

    lax.fori_loop(0, _NGRP, group, 0)
    pltpu.make_async_copy(tab_hbm.at[pl.ds(0, _BPW)], rows_v, sem).wait()
    pltpu.sync_copy(rows_v, out_hbm.at[pl.ds(base, _BPW)])


@functools.lru_cache(maxsize=1)
def _sc_gather_tail():
    return pl.kernel(
        _sc_rowdma_tail_body,
        out_type=jax.ShapeDtypeStruct((B, D), jnp.float32),
        mesh=plsc.VectorSubcoreMesh(core_axis_name="c", subcore_axis_name="s"),
        scratch_types=[
            pltpu.VMEM((_BPW,), jnp.int32),
            pltpu.VMEM((_BPW, D), jnp.float32),
            pltpu.SemaphoreType.DMA,
        ],
    )


@functools.lru_cache(maxsize=1)
def _sc_gather():
    return pl.kernel(
        _sc_rowdma_body,
        out_type=jax.ShapeDtypeStruct((B, D), jnp.float32),
        mesh=plsc.VectorSubcoreMesh(core_axis_name="c", subcore_axis_name="s"),
        scratch_types=[
            pltpu.VMEM((_BPW,), jnp.int32),
            pltpu.VMEM((_BPW, D), jnp.float32),
            pltpu.SemaphoreType.DMA,
        ],
    )


_BLK = 2048


def _mlp_body(u_ref, v_ref, w1u_ref, w1v_ref, b_ref, w2_ref, o_ref):
    h = (jnp.dot(u_ref[...], w1u_ref[...],
                 preferred_element_type=jnp.float32,
                 precision=lax.Precision.HIGHEST)
         + jnp.dot(v_ref[...], w1v_ref[...],
                   preferred_element_type=jnp.float32,
                   precision=lax.Precision.HIGHEST)
         + b_ref[...])
    h = jnp.maximum(h, 0.0)
    o_ref[...] = jnp.dot(h, w2_ref[...],
                         preferred_element_type=jnp.float32,
                         precision=lax.Precision.HIGHEST)


_mlp = pl.pallas_call(
    _mlp_body,
    grid=(B // _BLK,),
    in_specs=[
        pl.BlockSpec((_BLK, D), lambda i: (i, 0)),
        pl.BlockSpec((_BLK, D), lambda i: (i, 0)),
        pl.BlockSpec((D, D), lambda i: (0, 0)),
        pl.BlockSpec((D, D), lambda i: (0, 0)),
        pl.BlockSpec((1, D), lambda i: (0, 0)),
        pl.BlockSpec((D, 1), lambda i: (0, 0)),
    ],
    out_specs=pl.BlockSpec((_BLK, 1), lambda i: (i, 0)),
    out_shape=jax.ShapeDtypeStruct((B, 1), jnp.float32),
)


def kernel(x, W_table, H_table, lin1_w, lin1_b, lin2_w):
    uidx = x[:, 0]
    iidx = x[:, 1]
    h_rm = _sc_transpose()(H_table.T)
    h_tail = H_table[_NFULL * 128:]
    v_emb = _sc_gather_tail()(iidx, h_rm, h_tail)
    u_emb = _sc_gather()(uidx, W_table)
    w1t = lin1_w.T
    return _mlp(u_emb, v_emb, w1t[:D], w1t[D:], lin1_b.reshape(1, D),
                lin2_w.T)
